# Initial kernel scaffold; baseline (speedup 1.0000x reference)
import jax
import jax.numpy as jnp
from jax.experimental import pallas as pl


def kernel(state, lidar, aux, action_expert, W1, b1, W2, b2, W3, b3, cluster_weight, cluster_bias):
    raise NotImplementedError("write your pallas kernel here")


# speedup vs baseline: 5.6780x; 5.6780x over previous
import functools

import jax
import jax.numpy as jnp
from jax.experimental import pallas as pl

B = 16384
STATE_DIM = 35
LIDAR_DIM = 240
HIDDEN = 64
K = 5
ACT = 2
BLOCK_B = 2048


def _fused_body(xs_ref, xl_ref, ae_ref,
                w1s_ref, w1l_ref, b1_ref, w2_ref, b2_ref, w3_ref, b3_ref,
                wa0s_ref, wa0l_ref, wa1s_ref, wa1l_ref, cb0_ref, cb1_ref,
                act_ref, loss_ref):
    i = pl.program_id(0)
    xs = xs_ref[...]
    xl = xl_ref[...]

    f32 = jnp.float32

    def elu(x):
        return jnp.where(x > 0, x, jnp.exp(jnp.minimum(x, 0.0)) - 1.0)

    h = jnp.dot(xs, w1s_ref[...], preferred_element_type=f32)
    h = h + jnp.dot(xl, w1l_ref[...], preferred_element_type=f32)
    h = elu(h + b1_ref[...])
    h = elu(jnp.dot(h, w2_ref[...], preferred_element_type=f32) + b2_ref[...])
    z = jnp.dot(h, w3_ref[...], preferred_element_type=f32) + b3_ref[...]

    m = jnp.max(z, axis=1, keepdims=True)
    iota = jax.lax.broadcasted_iota(jnp.int32, z.shape, 1)
    cid = jnp.min(jnp.where(z == m, iota, K), axis=1, keepdims=True)
    onehot = (iota == cid).astype(f32)

    a0 = jnp.dot(xs, wa0s_ref[...], preferred_element_type=f32)
    a0 = a0 + jnp.dot(xl, wa0l_ref[...], preferred_element_type=f32)
    a1 = jnp.dot(xs, wa1s_ref[...], preferred_element_type=f32)
    a1 = a1 + jnp.dot(xl, wa1l_ref[...], preferred_element_type=f32)

    act0 = jnp.sum(onehot * (a0 + cb0_ref[...]), axis=1, keepdims=True)
    act1 = jnp.sum(onehot * (a1 + cb1_ref[...]), axis=1, keepdims=True)
    act_ref[:, 0:1] = act0
    act_ref[:, 1:2] = act1

    d0 = act0 - ae_ref[:, 0:1]
    d1 = act1 - ae_ref[:, 1:2]
    part = (jnp.sum(d0 * d0, axis=0, keepdims=True)
            + jnp.sum(d1 * d1, axis=0, keepdims=True)) * (1.0 / (B * ACT))

    @pl.when(i == 0)
    def _():
        loss_ref[...] = part

    @pl.when(i != 0)
    def _():
        loss_ref[...] = loss_ref[...] + part


@functools.partial(jax.jit, static_argnames=())
def kernel(state, lidar, aux, action_expert, W1, b1, W2, b2, W3, b3, cluster_weight, cluster_bias):
    del aux
    w1s, w1l = W1[:STATE_DIM], W1[STATE_DIM:]
    wa = jnp.transpose(cluster_weight, (1, 0, 2))
    wa0s, wa0l = wa[:STATE_DIM, :, 0], wa[STATE_DIM:, :, 0]
    wa1s, wa1l = wa[:STATE_DIM, :, 1], wa[STATE_DIM:, :, 1]
    b1r = b1.reshape(1, HIDDEN)
    b2r = b2.reshape(1, HIDDEN)
    b3r = b3.reshape(1, K)
    cb0 = cluster_bias[:, 0].reshape(1, K)
    cb1 = cluster_bias[:, 1].reshape(1, K)

    nblk = B // BLOCK_B
    row_spec = lambda cols: pl.BlockSpec((BLOCK_B, cols), lambda i: (i, 0))
    full = lambda shape: pl.BlockSpec(shape, lambda i: (0,) * len(shape))

    act, loss = pl.pallas_call(
        _fused_body,
        grid=(nblk,),
        in_specs=[
            row_spec(STATE_DIM),
            row_spec(LIDAR_DIM),
            row_spec(ACT),
            full((STATE_DIM, HIDDEN)),
            full((LIDAR_DIM, HIDDEN)),
            full((1, HIDDEN)),
            full((HIDDEN, HIDDEN)),
            full((1, HIDDEN)),
            full((HIDDEN, K)),
            full((1, K)),
            full((STATE_DIM, K)),
            full((LIDAR_DIM, K)),
            full((STATE_DIM, K)),
            full((LIDAR_DIM, K)),
            full((1, K)),
            full((1, K)),
        ],
        out_specs=[
            pl.BlockSpec((BLOCK_B, ACT), lambda i: (i, 0)),
            pl.BlockSpec((1, 1), lambda i: (0, 0)),
        ],
        out_shape=[
            jax.ShapeDtypeStruct((B, ACT), jnp.float32),
            jax.ShapeDtypeStruct((1, 1), jnp.float32),
        ],
    )(state, lidar, action_expert,
      w1s, w1l, b1r, W2, b2r, W3, b3r,
      wa0s, wa0l, wa1s, wa1l, cb0, cb1)
    return act, loss[0, 0]
